# baseline (speedup 1.0000x reference)
import functools

import jax
import jax.numpy as jnp
from jax import lax
from jax.experimental import pallas as pl
from jax.experimental.pallas import tpu as pltpu
from jax.experimental.pallas import tpu_sc as plsc

_K = 16
_N = 100
_SIGMA = 0.05
_B = 16
_D = 2048
_L = 16
_CHAINS = 4
_CHUNKS = _D // _L
_CPC = _CHUNKS // _CHAINS


def _bitonic_merge_desc(av, ai, bv, bi):
    rbv = lax.rev(bv, (0,))
    rbi = lax.rev(bi, (0,))
    m = av >= rbv
    nv = jnp.where(m, av, rbv)
    ni = jnp.where(m, ai, rbi)
    return plsc.sort_key_val(nv, ni, descending=True)


def _sc_kernel(x_hbm, noise_hbm, out_hbm, xrow, nb0, nb1, acc, sem0, sem1):
    c = lax.axis_index("c")
    s = lax.axis_index("s")
    b = s

    @pl.when(c == 0)
    def _():
        pltpu.sync_copy(x_hbm.at[b], xrow)
        zero16 = jnp.zeros((_L,), jnp.float32)
        for k in range(_K):
            def zbody(j, _, k=k):
                acc[k, pl.ds(j * _L, _L)] = zero16
                return 0
            lax.fori_loop(0, _D // _L, zbody, 0)

        iota = lax.iota(jnp.int32, _L)
        neg_inf = jnp.full((_L,), -jnp.inf, jnp.float32)
        zeros_i = jnp.zeros((_L,), jnp.int32)
        add_val = jnp.full((_L,), 1.0 / _N, jnp.float32)

        def process_row(nbuf):
            def chunk_body(j, carry):
                new = []
                for ch in range(_CHAINS):
                    tv, ti = carry[2 * ch], carry[2 * ch + 1]
                    d0 = (ch * _CPC + j) * _L
                    w = xrow[pl.ds(d0, _L)] + nbuf[pl.ds(d0, _L)] * _SIGMA
                    gi = iota + d0
                    sw, si = plsc.sort_key_val(w, gi, descending=False)
                    m = tv >= sw
                    nv = jnp.where(m, tv, sw)
                    ni = jnp.where(m, ti, si)
                    tv, ti = plsc.sort_key_val(nv, ni, descending=True)
                    new.extend((tv, ti))
                return tuple(new)

            init = (neg_inf, zeros_i) * _CHAINS
            res = lax.fori_loop(0, _CPC, chunk_body, init)
            v01, i01 = _bitonic_merge_desc(res[0], res[1], res[2], res[3])
            v23, i23 = _bitonic_merge_desc(res[4], res[5], res[6], res[7])
            tv, ti = _bitonic_merge_desc(v01, i01, v23, i23)
            plsc.addupdate_scatter(acc, [iota, ti], add_val)

        pltpu.make_async_copy(noise_hbm.at[b, 0], nb0, sem0).start()
        pltpu.make_async_copy(noise_hbm.at[b, 1], nb1, sem1).start()

        def row_body(rp, _):
            pltpu.make_async_copy(noise_hbm.at[b, 2 * rp], nb0, sem0).wait()
            process_row(nb0)

            @pl.when(rp < _N // 2 - 1)
            def _():
                pltpu.make_async_copy(
                    noise_hbm.at[b, 2 * rp + 2], nb0, sem0).start()

            pltpu.make_async_copy(noise_hbm.at[b, 2 * rp + 1], nb1, sem1).wait()
            process_row(nb1)

            @pl.when(rp < _N // 2 - 1)
            def _():
                pltpu.make_async_copy(
                    noise_hbm.at[b, 2 * rp + 3], nb1, sem1).start()

            return 0

        lax.fori_loop(0, _N // 2, row_body, 0)
        pltpu.sync_copy(acc, out_hbm.at[b])


@functools.lru_cache(maxsize=2)
def _fixed_noise(b, d):
    return jax.random.normal(
        jax.random.key(1), (b, _N, d), dtype=jnp.float32)


@functools.partial(jax.jit, static_argnames=())
def kernel(x):
    b, d = x.shape
    noise = _fixed_noise(b, d)
    mesh = plsc.VectorSubcoreMesh(core_axis_name="c", subcore_axis_name="s")
    run = functools.partial(
        pl.kernel,
        mesh=mesh,
        out_type=jax.ShapeDtypeStruct((b, _K, d), jnp.float32),
        scratch_types=[
            pltpu.VMEM((d,), jnp.float32),
            pltpu.VMEM((d,), jnp.float32),
            pltpu.VMEM((d,), jnp.float32),
            pltpu.VMEM((_K, d), jnp.float32),
            pltpu.SemaphoreType.DMA,
            pltpu.SemaphoreType.DMA,
        ],
        compiler_params=pltpu.CompilerParams(needs_layout_passes=False),
    )(_sc_kernel)
    return run(x, noise)


# speedup vs baseline: 1.1365x; 1.1365x over previous
import functools

import jax
import jax.numpy as jnp
from jax import lax
from jax.experimental import pallas as pl
from jax.experimental.pallas import tpu as pltpu
from jax.experimental.pallas import tpu_sc as plsc

_K = 16
_N = 100
_SIGMA = 0.05
_B = 16
_D = 2048
_L = 16
_CHAINS = 8
_CHUNKS = _D // _L
_CPC = _CHUNKS // _CHAINS
_RPW = _N // 2


def _bitonic_merge_desc(av, ai, bv, bi):
    rbv = lax.rev(bv, (0,))
    rbi = lax.rev(bi, (0,))
    m = av >= rbv
    nv = jnp.where(m, av, rbv)
    ni = jnp.where(m, ai, rbi)
    return plsc.sort_key_val(nv, ni, descending=True)


def _sc_kernel(x_hbm, noise_hbm, out_hbm, part_hbm,
               xrow, nb0, nb1, acc, tmp, sem0, sem1):
    c = lax.axis_index("c")
    s = lax.axis_index("s")
    b = c * (_B // 2) + s // 2
    half = s % 2
    n0 = half * _RPW

    pltpu.sync_copy(x_hbm.at[b], xrow)
    zero16 = jnp.zeros((_L,), jnp.float32)
    for k in range(_K):
        def zbody(j, _, k=k):
            acc[k, pl.ds(j * _L, _L)] = zero16
            return 0
        lax.fori_loop(0, _D // _L, zbody, 0)

    iota = lax.iota(jnp.int32, _L)
    neg_inf = jnp.full((_L,), -jnp.inf, jnp.float32)
    zeros_i = jnp.zeros((_L,), jnp.int32)
    add_val = jnp.full((_L,), 1.0 / _N, jnp.float32)

    def process_row(nbuf):
        def chunk_body(j, carry):
            new = []
            for ch in range(_CHAINS):
                tv, ti = carry[2 * ch], carry[2 * ch + 1]
                d0 = (ch * _CPC + j) * _L
                w = xrow[pl.ds(d0, _L)] + nbuf[pl.ds(d0, _L)] * _SIGMA
                gi = iota + d0
                sw, si = plsc.sort_key_val(w, gi, descending=False)
                m = tv >= sw
                nv = jnp.where(m, tv, sw)
                ni = jnp.where(m, ti, si)
                tv, ti = plsc.sort_key_val(nv, ni, descending=True)
                new.extend((tv, ti))
            return tuple(new)

        init = (neg_inf, zeros_i) * _CHAINS
        res = lax.fori_loop(0, _CPC, chunk_body, init)
        pairs = [(res[2 * i], res[2 * i + 1]) for i in range(_CHAINS)]
        while len(pairs) > 1:
            pairs = [_bitonic_merge_desc(*pairs[i], *pairs[i + 1])
                     for i in range(0, len(pairs), 2)]
        _, ti = pairs[0]
        plsc.addupdate_scatter(acc, [iota, ti], add_val)

    pltpu.make_async_copy(noise_hbm.at[b, n0], nb0, sem0).start()
    pltpu.make_async_copy(noise_hbm.at[b, n0 + 1], nb1, sem1).start()

    def row_body(rp, _):
        pltpu.make_async_copy(noise_hbm.at[b, n0 + 2 * rp], nb0, sem0).wait()
        process_row(nb0)

        @pl.when(rp < _RPW // 2 - 1)
        def _():
            pltpu.make_async_copy(
                noise_hbm.at[b, n0 + 2 * rp + 2], nb0, sem0).start()

        pltpu.make_async_copy(
            noise_hbm.at[b, n0 + 2 * rp + 1], nb1, sem1).wait()
        process_row(nb1)

        @pl.when(rp < _RPW // 2 - 1)
        def _():
            pltpu.make_async_copy(
                noise_hbm.at[b, n0 + 2 * rp + 3], nb1, sem1).start()

        return 0

    lax.fori_loop(0, _RPW // 2, row_body, 0)

    @pl.when(half == 1)
    def _():
        pltpu.sync_copy(acc, part_hbm.at[b])

    plsc.subcore_barrier()

    @pl.when(half == 0)
    def _():
        pltpu.sync_copy(part_hbm.at[b], tmp)
        for k in range(_K):
            def abody(j, _, k=k):
                sl = pl.ds(j * _L, _L)
                acc[k, sl] = acc[k, sl] + tmp[k, sl]
                return 0
            lax.fori_loop(0, _D // _L, abody, 0)
        pltpu.sync_copy(acc, out_hbm.at[b])


@functools.lru_cache(maxsize=2)
def _fixed_noise(b, d):
    return jax.random.normal(
        jax.random.key(1), (b, _N, d), dtype=jnp.float32)


@functools.partial(jax.jit, static_argnames=())
def kernel(x):
    b, d = x.shape
    noise = _fixed_noise(b, d)
    mesh = plsc.VectorSubcoreMesh(core_axis_name="c", subcore_axis_name="s")
    run = functools.partial(
        pl.kernel,
        mesh=mesh,
        out_type=(
            jax.ShapeDtypeStruct((b, _K, d), jnp.float32),
            jax.ShapeDtypeStruct((b, _K, d), jnp.float32),
        ),
        scratch_types=[
            pltpu.VMEM((d,), jnp.float32),
            pltpu.VMEM((d,), jnp.float32),
            pltpu.VMEM((d,), jnp.float32),
            pltpu.VMEM((_K, d), jnp.float32),
            pltpu.VMEM((_K, d), jnp.float32),
            pltpu.SemaphoreType.DMA,
            pltpu.SemaphoreType.DMA,
        ],
        compiler_params=pltpu.CompilerParams(needs_layout_passes=False),
    )(_sc_kernel)
    out, _ = run(x, noise)
    return out


# speedup vs baseline: 1.2429x; 1.0936x over previous
import functools

import jax
import jax.numpy as jnp
from jax import lax
from jax.experimental import pallas as pl
from jax.experimental.pallas import tpu as pltpu
from jax.experimental.pallas import tpu_sc as plsc

_K = 16
_N = 100
_SIGMA = 0.05
_B = 16
_D = 2048
_L = 16
_CHAINS = 8
_CHUNKS = _D // _L
_CPC = _CHUNKS // _CHAINS
_RPW = _N // 2


def _bitonic_merge_desc(av, ai, bv, bi):
    rbv = lax.rev(bv, (0,))
    rbi = lax.rev(bi, (0,))
    m = av >= rbv
    nv = jnp.where(m, av, rbv)
    ni = jnp.where(m, ai, rbi)
    return plsc.sort_key_val(nv, ni, descending=True)


def _sc_kernel(x_hbm, noise_hbm, out_hbm, part_hbm,
               xrow, nb0, nb1, acc, tmp, sem0, sem1):
    c = lax.axis_index("c")
    s = lax.axis_index("s")
    b = c * (_B // 2) + s // 2
    half = s % 2
    n0 = half * _RPW

    pltpu.sync_copy(x_hbm.at[b], xrow)
    zero16 = jnp.zeros((_L,), jnp.float32)
    for k in range(_K):
        def zbody(j, _, k=k):
            for jj in range(8):
                acc[k, pl.ds((j * 8 + jj) * _L, _L)] = zero16
            return 0
        lax.fori_loop(0, _D // (_L * 8), zbody, 0)

    iota = lax.iota(jnp.int32, _L)
    neg_inf = jnp.full((_L,), -jnp.inf, jnp.float32)
    zeros_i = jnp.zeros((_L,), jnp.int32)
    add_val = jnp.full((_L,), 1.0 / _N, jnp.float32)

    def process_row(nbuf):
        def chunk_body(j, carry):
            new = []
            for ch in range(_CHAINS):
                tv, ti = carry[2 * ch], carry[2 * ch + 1]
                d0 = (ch * _CPC + j) * _L
                w = xrow[pl.ds(d0, _L)] + nbuf[pl.ds(d0, _L)] * _SIGMA
                gi = iota + d0
                sw, si = plsc.sort_key_val(w, gi, descending=False)
                m = tv >= sw
                nv = jnp.where(m, tv, sw)
                ni = jnp.where(m, ti, si)
                tv, ti = plsc.sort_key_val(nv, ni, descending=True)
                new.extend((tv, ti))
            return tuple(new)

        init = (neg_inf, zeros_i) * _CHAINS
        res = lax.fori_loop(0, _CPC, chunk_body, init)
        pairs = [(res[2 * i], res[2 * i + 1]) for i in range(_CHAINS)]
        while len(pairs) > 1:
            pairs = [_bitonic_merge_desc(*pairs[i], *pairs[i + 1])
                     for i in range(0, len(pairs), 2)]
        _, ti = pairs[0]
        plsc.addupdate_scatter(acc, [iota, ti], add_val)

    pltpu.make_async_copy(noise_hbm.at[b, n0], nb0, sem0).start()
    pltpu.make_async_copy(noise_hbm.at[b, n0 + 1], nb1, sem1).start()

    def row_body(rp, _):
        pltpu.make_async_copy(noise_hbm.at[b, n0 + 2 * rp], nb0, sem0).wait()
        process_row(nb0)

        @pl.when(rp < _RPW // 2 - 1)
        def _():
            pltpu.make_async_copy(
                noise_hbm.at[b, n0 + 2 * rp + 2], nb0, sem0).start()

        pltpu.make_async_copy(
            noise_hbm.at[b, n0 + 2 * rp + 1], nb1, sem1).wait()
        process_row(nb1)

        @pl.when(rp < _RPW // 2 - 1)
        def _():
            pltpu.make_async_copy(
                noise_hbm.at[b, n0 + 2 * rp + 3], nb1, sem1).start()

        return 0

    lax.fori_loop(0, _RPW // 2, row_body, 0)

    @pl.when(half == 1)
    def _():
        pltpu.sync_copy(acc, part_hbm.at[b])

    plsc.subcore_barrier()

    @pl.when(half == 0)
    def _():
        pltpu.sync_copy(part_hbm.at[b], tmp)
        for k in range(_K):
            def abody(j, _, k=k):
                for jj in range(8):
                    sl = pl.ds((j * 8 + jj) * _L, _L)
                    acc[k, sl] = acc[k, sl] + tmp[k, sl]
                return 0
            lax.fori_loop(0, _D // (_L * 8), abody, 0)
        pltpu.sync_copy(acc, out_hbm.at[b])


@functools.lru_cache(maxsize=2)
def _fixed_noise(b, d):
    return jax.random.normal(
        jax.random.key(1), (b, _N, d), dtype=jnp.float32)


@functools.partial(jax.jit, static_argnames=())
def kernel(x):
    b, d = x.shape
    noise = _fixed_noise(b, d)
    mesh = plsc.VectorSubcoreMesh(core_axis_name="c", subcore_axis_name="s")
    run = functools.partial(
        pl.kernel,
        mesh=mesh,
        out_type=(
            jax.ShapeDtypeStruct((b, _K, d), jnp.float32),
            jax.ShapeDtypeStruct((b, _K, d), jnp.float32),
        ),
        scratch_types=[
            pltpu.VMEM((d,), jnp.float32),
            pltpu.VMEM((d,), jnp.float32),
            pltpu.VMEM((d,), jnp.float32),
            pltpu.VMEM((_K, d), jnp.float32),
            pltpu.VMEM((_K, d), jnp.float32),
            pltpu.SemaphoreType.DMA,
            pltpu.SemaphoreType.DMA,
        ],
        compiler_params=pltpu.CompilerParams(needs_layout_passes=False),
    )(_sc_kernel)
    out, _ = run(x, noise)
    return out
